# Initial kernel scaffold; baseline (speedup 1.0000x reference)
import jax
import jax.numpy as jnp
from jax.experimental import pallas as pl


def kernel(p, x, o, target):
    raise NotImplementedError("write your pallas kernel here")


# speedup vs baseline: 16.9651x; 16.9651x over previous
import functools

import jax
import jax.numpy as jnp
from jax import lax
from jax.experimental import pallas as pl

_N = 8192
_C = 64
_K = 8
_EPS = 1e-12
_R = 128

_INTERPRET = False


def _normalize_body(x_ref, f_ref):
    x = x_ref[...]
    norm = jnp.sqrt(jnp.sum(x * x, axis=-1, keepdims=True))
    f_ref[...] = x / jnp.maximum(norm, _EPS)


def _knn_body(pp_ref, ppt_ref, f_ref, ft_ref, tcol_ref, trow_ref,
              g_ref, m_ref):
    pp = pp_ref[...]
    ppt = ppt_ref[...]
    pn_row = jnp.sum(ppt * ppt, axis=0, keepdims=True)
    pn_col = jnp.sum(pp * pp, axis=1, keepdims=True)
    dot = jnp.dot(pp, ppt, preferred_element_type=jnp.float32)
    d2 = pn_col + pn_row - 2.0 * dot

    gram = jnp.dot(f_ref[...], ft_ref[...],
                   preferred_element_type=jnp.float32)
    match = (tcol_ref[...] == trow_ref[...]).astype(jnp.float32)

    iota = lax.broadcasted_iota(jnp.int32, (_R, _N), 1)
    cur = d2
    gs = []
    ms = []
    for _ in range(_K):
        mn = jnp.min(cur, axis=1, keepdims=True)
        is_min = cur == mn
        idxk = jnp.min(jnp.where(is_min, iota, jnp.int32(2**30)),
                       axis=1, keepdims=True)
        onehot = iota == idxk
        gs.append(jnp.sum(jnp.where(onehot, gram, 0.0), axis=1,
                          keepdims=True))
        ms.append(jnp.sum(jnp.where(onehot, match, 0.0), axis=1,
                          keepdims=True))
        cur = jnp.where(onehot, jnp.float32(1e30), cur)
    g_ref[...] = jnp.concatenate(gs, axis=1)
    m_ref[...] = jnp.concatenate(ms, axis=1)


def _loss_body(g_ref, m_ref, out_ref):
    g = g_ref[...]
    pos = m_ref[...]
    col = lax.broadcasted_iota(jnp.int32, (_N, _K), 1)
    real = col >= 1

    sumsq = jnp.maximum(2.0 - 2.0 * g, 0.0)
    dist = jnp.sqrt(sumsq + _EPS)
    d = jnp.where(real, -dist, jnp.float32(-1e30))
    d = d - jnp.max(d, axis=1, keepdims=True)
    e = jnp.where(real, jnp.exp(d), 0.0)
    pos = jnp.where(real, pos, 0.0)
    negs = jnp.sum(e * (1.0 - pos), axis=1, keepdims=True)
    under = e + negs
    cnt = jnp.sum(pos, axis=1, keepdims=True)
    point_mask = jnp.logical_and(cnt > 0.5, cnt < jnp.float32(_K - 1) - 0.5)
    valid = jnp.logical_and(pos > 0.5, point_mask)
    l = -jnp.log(e / under)
    lsum = jnp.sum(jnp.where(valid, l, 0.0), keepdims=True)
    vsum = jnp.sum(valid.astype(jnp.float32), keepdims=True)
    out_ref[...] = lsum / jnp.maximum(vsum, 1.0)


@jax.jit
def kernel(p, x, o, target):
    del o
    feats = pl.pallas_call(
        _normalize_body,
        out_shape=jax.ShapeDtypeStruct((_N, _C), jnp.float32),
        interpret=_INTERPRET,
    )(x)
    ft = feats.T
    pp = jnp.pad(p, ((0, 0), (0, 5)))
    ppt = pp.T
    tcol = target.reshape(_N, 1)
    trow = target.reshape(1, _N)

    nblk = _N // _R
    g_tab, m_tab = pl.pallas_call(
        _knn_body,
        grid=(nblk,),
        in_specs=[
            pl.BlockSpec((_R, 8), lambda i: (i, 0)),
            pl.BlockSpec((8, _N), lambda i: (0, 0)),
            pl.BlockSpec((_R, _C), lambda i: (i, 0)),
            pl.BlockSpec((_C, _N), lambda i: (0, 0)),
            pl.BlockSpec((_R, 1), lambda i: (i, 0)),
            pl.BlockSpec((1, _N), lambda i: (0, 0)),
        ],
        out_specs=[
            pl.BlockSpec((_R, _K), lambda i: (i, 0)),
            pl.BlockSpec((_R, _K), lambda i: (i, 0)),
        ],
        out_shape=[
            jax.ShapeDtypeStruct((_N, _K), jnp.float32),
            jax.ShapeDtypeStruct((_N, _K), jnp.float32),
        ],
        interpret=_INTERPRET,
    )(pp, ppt, feats, ft, tcol, trow)

    out = pl.pallas_call(
        _loss_body,
        out_shape=jax.ShapeDtypeStruct((1, 1), jnp.float32),
        interpret=_INTERPRET,
    )(g_tab, m_tab)
    return out[0, 0]


# speedup vs baseline: 25.6837x; 1.5139x over previous
import functools

import jax
import jax.numpy as jnp
from jax import lax
from jax.experimental import pallas as pl
from jax.experimental.pallas import tpu as pltpu
from jax.experimental.pallas import tpu_sc as plsc

_N = 8192
_C = 64
_CP = 128
_K = 8
_KN = _K - 1
_EPS = 1e-12
_R = 128

_NW = 32
_QSUB = 16
_PSUB = _QSUB * _KN
_NSUB = _N // _NW // _QSUB
_P = _N * _KN

_INTERPRET = False


def _normalize_body(x_ref, f_ref):
    x = x_ref[...]
    norm = jnp.sqrt(jnp.sum(x * x, axis=-1, keepdims=True))
    f_ref[...] = x / jnp.maximum(norm, _EPS)


def _knn_body(pp_ref, ppt_ref, i_ref):
    pp = pp_ref[...]
    ppt = ppt_ref[...]
    pn_row = jnp.sum(ppt * ppt, axis=0, keepdims=True)
    pn_col = jnp.sum(pp * pp, axis=1, keepdims=True)
    dot = jnp.dot(pp, ppt, preferred_element_type=jnp.float32)
    cur = pn_col + pn_row - 2.0 * dot

    iota = lax.broadcasted_iota(jnp.int32, (_R, _N), 1)
    idxs = []
    for _ in range(_K):
        mn = jnp.min(cur, axis=1, keepdims=True)
        is_min = cur == mn
        idxk = jnp.min(jnp.where(is_min, iota, jnp.int32(2**30)),
                       axis=1, keepdims=True)
        idxs.append(idxk)
        cur = jnp.where(iota == idxk, jnp.float32(1e30), cur)
    i_ref[...] = jnp.concatenate(idxs[1:], axis=1)


def _pairs_body(feats_hbm, idx_hbm, tgt_hbm, sumsq_hbm, match_hbm,
                nidx_v, nf_v, qf_v, tgt_v, acc_v, out_v, mout_v, sem):
    wid = lax.axis_index("s") * 2 + lax.axis_index("c")
    qbase = wid * (_N // _NW)
    pltpu.sync_copy(tgt_hbm, tgt_v)
    pltpu.sync_copy(idx_hbm.at[pl.ds(wid * _NSUB, _NSUB)], nidx_v)
    lane = lax.broadcasted_iota(jnp.int32, (16,), 0)

    def sub_chunk(s, carry):
        q0 = qbase + s * _QSUB
        p0 = (q0 * _KN).astype(jnp.int32)
        pltpu.async_copy(feats_hbm.at[nidx_v.at[s]], nf_v, sem).wait()
        pltpu.sync_copy(feats_hbm.at[pl.ds(q0, _QSUB)], qf_v)
        for q in range(_QSUB):
            for j in range(_KN):
                r = q * _KN + j
                acc = jnp.zeros((16,), jnp.float32)
                for c in range(_C // 16):
                    d = (qf_v[q, pl.ds(c * 16, 16)]
                         - nf_v[r, pl.ds(c * 16, 16)])
                    acc = acc + d * d
                acc_v[pl.ds(r * 16, 16)] = acc
        for g in range(_PSUB // 16):
            tot = jnp.zeros((16,), jnp.float32)
            for c in range(16):
                tot = tot + plsc.load_gather(
                    acc_v,
                    [(jnp.full((16,), g * 16, jnp.int32) + lane) * 16
                     + jnp.full((16,), c, jnp.int32)])
            out_v[pl.ds(g * 16, 16)] = tot
            lp = jnp.full((16,), g * 16, jnp.int32) + lane
            nid = nidx_v[s, pl.ds(g * 16, 16)]
            tn = plsc.load_gather(tgt_v, [nid])
            tq = plsc.load_gather(tgt_v, [q0 + lp // _KN])
            mout_v[pl.ds(g * 16, 16)] = jnp.where(
                tq == tn, jnp.float32(1.0), jnp.float32(0.0))
        pltpu.sync_copy(out_v, sumsq_hbm.at[pl.ds(p0, _PSUB)])
        pltpu.sync_copy(mout_v, match_hbm.at[pl.ds(p0, _PSUB)])
        return carry

    lax.fori_loop(0, _NSUB, sub_chunk, 0)


def _loss_body(s_ref, m_ref, out_ref):
    sumsq = s_ref[...]
    pos = m_ref[...]
    dist = jnp.sqrt(sumsq + _EPS)
    d = -dist
    d = d - jnp.max(d, axis=1, keepdims=True)
    e = jnp.exp(d)
    negs = jnp.sum(e * (1.0 - pos), axis=1, keepdims=True)
    under = e + negs
    cnt = jnp.sum(pos, axis=1, keepdims=True)
    point_mask = jnp.logical_and(cnt > 0.5, cnt < jnp.float32(_KN) - 0.5)
    valid = jnp.logical_and(pos > 0.5, point_mask)
    l = -jnp.log(e / under)
    lsum = jnp.sum(jnp.where(valid, l, 0.0), keepdims=True)
    vsum = jnp.sum(valid.astype(jnp.float32), keepdims=True)
    out_ref[...] = lsum / jnp.maximum(vsum, 1.0)


@jax.jit
def kernel(p, x, o, target):
    del o
    feats = pl.pallas_call(
        _normalize_body,
        out_shape=jax.ShapeDtypeStruct((_N, _CP), jnp.float32),
        interpret=_INTERPRET,
    )(jnp.pad(x, ((0, 0), (0, _CP - _C))))
    pp = jnp.pad(p, ((0, 0), (0, 5)))
    ppt = pp.T

    nblk = _N // _R
    idx_tab = pl.pallas_call(
        _knn_body,
        grid=(nblk,),
        in_specs=[
            pl.BlockSpec((_R, 8), lambda i: (i, 0)),
            pl.BlockSpec((8, _N), lambda i: (0, 0)),
        ],
        out_specs=pl.BlockSpec((_R, _KN), lambda i: (i, 0)),
        out_shape=jax.ShapeDtypeStruct((_N, _KN), jnp.int32),
        interpret=_INTERPRET,
    )(pp, ppt)

    idx2d = idx_tab.reshape(_NW * _NSUB, _PSUB)
    pairs = pl.kernel(
        _pairs_body,
        out_type=[
            jax.ShapeDtypeStruct((_P,), jnp.float32),
            jax.ShapeDtypeStruct((_P,), jnp.float32),
        ],
        mesh=plsc.VectorSubcoreMesh(core_axis_name="c", subcore_axis_name="s"),
        compiler_params=pltpu.CompilerParams(needs_layout_passes=False),
        scratch_types=[
            pltpu.VMEM((_NSUB, _PSUB), jnp.int32),
            pltpu.VMEM((_PSUB, _CP), jnp.float32),
            pltpu.VMEM((_QSUB, _CP), jnp.float32),
            pltpu.VMEM((_N,), jnp.int32),
            pltpu.VMEM((_PSUB * 16,), jnp.float32),
            pltpu.VMEM((_PSUB,), jnp.float32),
            pltpu.VMEM((_PSUB,), jnp.float32),
            pltpu.SemaphoreType.DMA,
        ],
    )
    sumsq_flat, match_flat = pairs(feats, idx2d, target)

    out = pl.pallas_call(
        _loss_body,
        out_shape=jax.ShapeDtypeStruct((1, 1), jnp.float32),
        interpret=_INTERPRET,
    )(sumsq_flat.reshape(_N, _KN), match_flat.reshape(_N, _KN))
    return out[0, 0]


# speedup vs baseline: 30.5972x; 1.1913x over previous
import functools

import jax
import jax.numpy as jnp
from jax import lax
from jax.experimental import pallas as pl
from jax.experimental.pallas import tpu as pltpu
from jax.experimental.pallas import tpu_sc as plsc

_N = 8192
_C = 64
_CP = 128
_K = 8
_KN = _K - 1
_EPS = 1e-12
_R = 128

_NW = 32
_QSUB = 16
_PSUB = _QSUB * _KN
_NSUB = _N // _NW // _QSUB
_P = _N * _KN

_INTERPRET = False


def _normalize_body(x_ref, f_ref):
    x = x_ref[...]
    norm = jnp.sqrt(jnp.sum(x * x, axis=-1, keepdims=True))
    f_ref[...] = x / jnp.maximum(norm, _EPS)


def _knn_body(pp_ref, ppt_ref, i_ref):
    pp = pp_ref[...]
    ppt = ppt_ref[...]
    pn_row = jnp.sum(ppt * ppt, axis=0, keepdims=True)
    pn_col = jnp.sum(pp * pp, axis=1, keepdims=True)
    dot = jnp.dot(pp, ppt, preferred_element_type=jnp.float32)
    d2 = pn_col + pn_row - 2.0 * dot

    iota = lax.broadcasted_iota(jnp.int32, (_R, _N), 1).astype(jnp.float32)
    cur = d2
    idxs = []
    for _ in range(_K):
        mn = jnp.min(cur, axis=1, keepdims=True)
        idxf = jnp.min(jnp.where(cur == mn, iota, jnp.float32(3e7)),
                       axis=1, keepdims=True)
        idxs.append(idxf)
        cur = jnp.where(iota == idxf, jnp.float32(1e30), cur)
    i_ref[...] = jnp.concatenate(idxs[1:], axis=1).astype(jnp.int32)


def _pairs_body(feats_hbm, idx_hbm, tgt_hbm, sumsq_hbm, match_hbm,
                nidx_v, nf_v, qf_v, tgt_v, acc_v, out_v, mout_v, sem):
    wid = lax.axis_index("s") * 2 + lax.axis_index("c")
    qbase = wid * (_N // _NW)
    pltpu.sync_copy(tgt_hbm, tgt_v)
    pltpu.sync_copy(idx_hbm.at[pl.ds(wid * _NSUB, _NSUB)], nidx_v)
    lane = lax.broadcasted_iota(jnp.int32, (16,), 0)

    def sub_chunk(s, carry):
        q0 = qbase + s * _QSUB
        p0 = (q0 * _KN).astype(jnp.int32)
        pltpu.async_copy(feats_hbm.at[nidx_v.at[s]], nf_v, sem).wait()
        pltpu.sync_copy(feats_hbm.at[pl.ds(q0, _QSUB)], qf_v)
        for q in range(_QSUB):
            for j in range(_KN):
                r = q * _KN + j
                acc = jnp.zeros((16,), jnp.float32)
                for c in range(_C // 16):
                    d = (qf_v[q, pl.ds(c * 16, 16)]
                         - nf_v[r, pl.ds(c * 16, 16)])
                    acc = acc + d * d
                acc_v[pl.ds(r * 16, 16)] = acc
        for g in range(_PSUB // 16):
            tot = jnp.zeros((16,), jnp.float32)
            for c in range(16):
                tot = tot + plsc.load_gather(
                    acc_v,
                    [(jnp.full((16,), g * 16, jnp.int32) + lane) * 16
                     + jnp.full((16,), c, jnp.int32)])
            out_v[pl.ds(g * 16, 16)] = tot
            lp = jnp.full((16,), g * 16, jnp.int32) + lane
            nid = nidx_v[s, pl.ds(g * 16, 16)]
            tn = plsc.load_gather(tgt_v, [nid])
            tq = plsc.load_gather(tgt_v, [q0 + lp // _KN])
            mout_v[pl.ds(g * 16, 16)] = jnp.where(
                tq == tn, jnp.float32(1.0), jnp.float32(0.0))
        pltpu.sync_copy(out_v, sumsq_hbm.at[pl.ds(p0, _PSUB)])
        pltpu.sync_copy(mout_v, match_hbm.at[pl.ds(p0, _PSUB)])
        return carry

    lax.fori_loop(0, _NSUB, sub_chunk, 0)


def _loss_body(s_ref, m_ref, out_ref):
    sumsq = s_ref[...]
    pos = m_ref[...]
    dist = jnp.sqrt(sumsq + _EPS)
    d = -dist
    d = d - jnp.max(d, axis=1, keepdims=True)
    e = jnp.exp(d)
    negs = jnp.sum(e * (1.0 - pos), axis=1, keepdims=True)
    under = e + negs
    cnt = jnp.sum(pos, axis=1, keepdims=True)
    point_mask = jnp.logical_and(cnt > 0.5, cnt < jnp.float32(_KN) - 0.5)
    valid = jnp.logical_and(pos > 0.5, point_mask)
    l = -jnp.log(e / under)
    lsum = jnp.sum(jnp.where(valid, l, 0.0), keepdims=True)
    vsum = jnp.sum(valid.astype(jnp.float32), keepdims=True)
    out_ref[...] = lsum / jnp.maximum(vsum, 1.0)


@jax.jit
def kernel(p, x, o, target):
    del o
    feats = pl.pallas_call(
        _normalize_body,
        out_shape=jax.ShapeDtypeStruct((_N, _CP), jnp.float32),
        interpret=_INTERPRET,
    )(jnp.pad(x, ((0, 0), (0, _CP - _C))))
    pp = jnp.pad(p, ((0, 0), (0, 5)))
    ppt = pp.T

    nblk = _N // _R
    idx_tab = pl.pallas_call(
        _knn_body,
        grid=(nblk,),
        in_specs=[
            pl.BlockSpec((_R, 8), lambda i: (i, 0)),
            pl.BlockSpec((8, _N), lambda i: (0, 0)),
        ],
        out_specs=pl.BlockSpec((_R, _KN), lambda i: (i, 0)),
        out_shape=jax.ShapeDtypeStruct((_N, _KN), jnp.int32),
        interpret=_INTERPRET,
    )(pp, ppt)

    idx2d = idx_tab.reshape(_NW * _NSUB, _PSUB)
    pairs = pl.kernel(
        _pairs_body,
        out_type=[
            jax.ShapeDtypeStruct((_P,), jnp.float32),
            jax.ShapeDtypeStruct((_P,), jnp.float32),
        ],
        mesh=plsc.VectorSubcoreMesh(core_axis_name="c", subcore_axis_name="s"),
        compiler_params=pltpu.CompilerParams(needs_layout_passes=False),
        scratch_types=[
            pltpu.VMEM((_NSUB, _PSUB), jnp.int32),
            pltpu.VMEM((_PSUB, _CP), jnp.float32),
            pltpu.VMEM((_QSUB, _CP), jnp.float32),
            pltpu.VMEM((_N,), jnp.int32),
            pltpu.VMEM((_PSUB * 16,), jnp.float32),
            pltpu.VMEM((_PSUB,), jnp.float32),
            pltpu.VMEM((_PSUB,), jnp.float32),
            pltpu.SemaphoreType.DMA,
        ],
    )
    sumsq_flat, match_flat = pairs(feats, idx2d, target)

    out = pl.pallas_call(
        _loss_body,
        out_shape=jax.ShapeDtypeStruct((1, 1), jnp.float32),
        interpret=_INTERPRET,
    )(sumsq_flat.reshape(_N, _KN), match_flat.reshape(_N, _KN))
    return out[0, 0]


# speedup vs baseline: 33.5599x; 1.0968x over previous
import functools

import jax
import jax.numpy as jnp
from jax import lax
from jax.experimental import pallas as pl
from jax.experimental.pallas import tpu as pltpu
from jax.experimental.pallas import tpu_sc as plsc

_N = 8192
_C = 64
_CP = 128
_K = 8
_KN = _K - 1
_EPS = 1e-12
_R = 128

_NW = 32
_QSUB = 16
_PSUB = _QSUB * _KN
_NSUB = _N // _NW // _QSUB
_P = _N * _KN

_INTERPRET = False


def _normalize_body(x_ref, f_ref):
    x = x_ref[...]
    norm = jnp.sqrt(jnp.sum(x * x, axis=-1, keepdims=True))
    f_ref[...] = x / jnp.maximum(norm, _EPS)


def _knn_body(pp_ref, ppt_ref, i_ref):
    pp = pp_ref[...]
    ppt = ppt_ref[...]
    pn_row = jnp.sum(ppt * ppt, axis=0, keepdims=True)
    pn_col = jnp.sum(pp * pp, axis=1, keepdims=True)
    dot = jnp.dot(pp, ppt, preferred_element_type=jnp.float32)
    d2 = pn_col + pn_row - 2.0 * dot

    iota = lax.broadcasted_iota(jnp.int32, (_R, _N), 1)
    s = lax.bitcast_convert_type(d2, jnp.int32)
    t = s ^ (lax.shift_right_arithmetic(s, 31) & jnp.int32(0x7FFFFFFF))
    cur = (t & jnp.int32(-8192)) | iota
    keys = []
    for _ in range(_K):
        mn = jnp.min(cur, axis=1, keepdims=True)
        keys.append(mn)
        cur = jnp.where(cur == mn, jnp.int32(0x7FFFFFFF), cur)
    i_ref[...] = jnp.concatenate(keys[1:], axis=1) & jnp.int32(8191)


def _pairs_body(feats_hbm, idx_hbm, tgt_hbm, sumsq_hbm, match_hbm,
                nidx_v, nf_v, qf_v, tgt_v, acc_v, out_v, mout_v, sem):
    wid = lax.axis_index("s") * 2 + lax.axis_index("c")
    qbase = wid * (_N // _NW)
    pltpu.sync_copy(tgt_hbm, tgt_v)
    pltpu.sync_copy(idx_hbm.at[pl.ds(wid * _NSUB, _NSUB)], nidx_v)
    lane = lax.broadcasted_iota(jnp.int32, (16,), 0)

    def sub_chunk(s, carry):
        q0 = qbase + s * _QSUB
        p0 = (q0 * _KN).astype(jnp.int32)
        pltpu.async_copy(feats_hbm.at[nidx_v.at[s]], nf_v, sem).wait()
        pltpu.sync_copy(feats_hbm.at[pl.ds(q0, _QSUB)], qf_v)
        for q in range(_QSUB):
            for j in range(_KN):
                r = q * _KN + j
                acc = jnp.zeros((16,), jnp.float32)
                for c in range(_C // 16):
                    d = (qf_v[q, pl.ds(c * 16, 16)]
                         - nf_v[r, pl.ds(c * 16, 16)])
                    acc = acc + d * d
                acc_v[pl.ds(r * 16, 16)] = acc
        for g in range(_PSUB // 16):
            tot = jnp.zeros((16,), jnp.float32)
            for c in range(16):
                tot = tot + plsc.load_gather(
                    acc_v,
                    [(jnp.full((16,), g * 16, jnp.int32) + lane) * 16
                     + jnp.full((16,), c, jnp.int32)])
            out_v[pl.ds(g * 16, 16)] = tot
            lp = jnp.full((16,), g * 16, jnp.int32) + lane
            nid = nidx_v[s, pl.ds(g * 16, 16)]
            tn = plsc.load_gather(tgt_v, [nid])
            tq = plsc.load_gather(tgt_v, [q0 + lp // _KN])
            mout_v[pl.ds(g * 16, 16)] = jnp.where(
                tq == tn, jnp.float32(1.0), jnp.float32(0.0))
        pltpu.sync_copy(out_v, sumsq_hbm.at[pl.ds(p0, _PSUB)])
        pltpu.sync_copy(mout_v, match_hbm.at[pl.ds(p0, _PSUB)])
        return carry

    lax.fori_loop(0, _NSUB, sub_chunk, 0)


def _loss_body(s_ref, m_ref, out_ref):
    sumsq = s_ref[...]
    pos = m_ref[...]
    dist = jnp.sqrt(sumsq + _EPS)
    d = -dist
    d = d - jnp.max(d, axis=1, keepdims=True)
    e = jnp.exp(d)
    negs = jnp.sum(e * (1.0 - pos), axis=1, keepdims=True)
    under = e + negs
    cnt = jnp.sum(pos, axis=1, keepdims=True)
    point_mask = jnp.logical_and(cnt > 0.5, cnt < jnp.float32(_KN) - 0.5)
    valid = jnp.logical_and(pos > 0.5, point_mask)
    l = -jnp.log(e / under)
    lsum = jnp.sum(jnp.where(valid, l, 0.0), keepdims=True)
    vsum = jnp.sum(valid.astype(jnp.float32), keepdims=True)
    out_ref[...] = lsum / jnp.maximum(vsum, 1.0)


@jax.jit
def kernel(p, x, o, target):
    del o
    feats = pl.pallas_call(
        _normalize_body,
        out_shape=jax.ShapeDtypeStruct((_N, _CP), jnp.float32),
        interpret=_INTERPRET,
    )(jnp.pad(x, ((0, 0), (0, _CP - _C))))
    pp = jnp.pad(p, ((0, 0), (0, 5)))
    ppt = pp.T

    nblk = _N // _R
    idx_tab = pl.pallas_call(
        _knn_body,
        grid=(nblk,),
        in_specs=[
            pl.BlockSpec((_R, 8), lambda i: (i, 0)),
            pl.BlockSpec((8, _N), lambda i: (0, 0)),
        ],
        out_specs=pl.BlockSpec((_R, _KN), lambda i: (i, 0)),
        out_shape=jax.ShapeDtypeStruct((_N, _KN), jnp.int32),
        interpret=_INTERPRET,
    )(pp, ppt)

    idx2d = idx_tab.reshape(_NW * _NSUB, _PSUB)
    pairs = pl.kernel(
        _pairs_body,
        out_type=[
            jax.ShapeDtypeStruct((_P,), jnp.float32),
            jax.ShapeDtypeStruct((_P,), jnp.float32),
        ],
        mesh=plsc.VectorSubcoreMesh(core_axis_name="c", subcore_axis_name="s"),
        compiler_params=pltpu.CompilerParams(needs_layout_passes=False),
        scratch_types=[
            pltpu.VMEM((_NSUB, _PSUB), jnp.int32),
            pltpu.VMEM((_PSUB, _CP), jnp.float32),
            pltpu.VMEM((_QSUB, _CP), jnp.float32),
            pltpu.VMEM((_N,), jnp.int32),
            pltpu.VMEM((_PSUB * 16,), jnp.float32),
            pltpu.VMEM((_PSUB,), jnp.float32),
            pltpu.VMEM((_PSUB,), jnp.float32),
            pltpu.SemaphoreType.DMA,
        ],
    )
    sumsq_flat, match_flat = pairs(feats, idx2d, target)

    out = pl.pallas_call(
        _loss_body,
        out_shape=jax.ShapeDtypeStruct((1, 1), jnp.float32),
        interpret=_INTERPRET,
    )(sumsq_flat.reshape(_N, _KN), match_flat.reshape(_N, _KN))
    return out[0, 0]


# speedup vs baseline: 35.4321x; 1.0558x over previous
import functools

import jax
import jax.numpy as jnp
from jax import lax
from jax.experimental import pallas as pl
from jax.experimental.pallas import tpu as pltpu
from jax.experimental.pallas import tpu_sc as plsc

_N = 8192
_C = 64
_CP = 128
_K = 8
_KN = _K - 1
_EPS = 1e-12
_R = 256

_NW = 32
_QSUB = 16
_PSUB = _QSUB * _KN
_NSUB = _N // _NW // _QSUB
_P = _N * _KN

_INTERPRET = False


def _normalize_body(x_ref, f_ref):
    x = x_ref[...]
    norm = jnp.sqrt(jnp.sum(x * x, axis=-1, keepdims=True))
    f_ref[...] = x / jnp.maximum(norm, _EPS)


def _knn_body(pp_ref, ppt_ref, i_ref):
    pp = pp_ref[...]
    ppt = ppt_ref[...]
    pn_row = jnp.sum(ppt * ppt, axis=0, keepdims=True)
    pn_col = jnp.sum(pp * pp, axis=1, keepdims=True)
    dot = jnp.dot(pp, ppt, preferred_element_type=jnp.float32)
    d2 = pn_col + pn_row - 2.0 * dot

    iota = lax.broadcasted_iota(jnp.int32, (_R, _N), 1)
    s = lax.bitcast_convert_type(d2, jnp.int32)
    t = s ^ (lax.shift_right_arithmetic(s, 31) & jnp.int32(0x7FFFFFFF))
    cur = (t & jnp.int32(-8192)) | iota
    keys = []
    for _ in range(_K):
        mn = jnp.min(cur, axis=1, keepdims=True)
        keys.append(mn)
        cur = jnp.where(cur == mn, jnp.int32(0x7FFFFFFF), cur)
    i_ref[...] = jnp.concatenate(keys[1:], axis=1) & jnp.int32(8191)


def _pairs_body(feats_hbm, idx_hbm, tgt_hbm, sumsq_hbm, match_hbm,
                nidx_v, nf_v, qf_v, tgt_v, acc_v, out_v, mout_v, sem):
    wid = lax.axis_index("s") * 2 + lax.axis_index("c")
    qbase = wid * (_N // _NW)
    pltpu.sync_copy(tgt_hbm, tgt_v)
    pltpu.sync_copy(idx_hbm.at[pl.ds(wid * _NSUB, _NSUB)], nidx_v)
    lane = lax.broadcasted_iota(jnp.int32, (16,), 0)

    def sub_chunk(s, carry):
        q0 = qbase + s * _QSUB
        p0 = (q0 * _KN).astype(jnp.int32)
        pltpu.async_copy(feats_hbm.at[nidx_v.at[s]], nf_v, sem).wait()
        pltpu.sync_copy(feats_hbm.at[pl.ds(q0, _QSUB)], qf_v)
        for q in range(_QSUB):
            for j in range(_KN):
                r = q * _KN + j
                acc = jnp.zeros((16,), jnp.float32)
                for c in range(_C // 16):
                    d = (qf_v[q, pl.ds(c * 16, 16)]
                         - nf_v[r, pl.ds(c * 16, 16)])
                    acc = acc + d * d
                acc_v[pl.ds(r * 16, 16)] = acc
        for g in range(_PSUB // 16):
            tot = jnp.zeros((16,), jnp.float32)
            for c in range(16):
                tot = tot + plsc.load_gather(
                    acc_v,
                    [(jnp.full((16,), g * 16, jnp.int32) + lane) * 16
                     + jnp.full((16,), c, jnp.int32)])
            out_v[pl.ds(g * 16, 16)] = tot
            lp = jnp.full((16,), g * 16, jnp.int32) + lane
            nid = nidx_v[s, pl.ds(g * 16, 16)]
            tn = plsc.load_gather(tgt_v, [nid])
            tq = plsc.load_gather(tgt_v, [q0 + lp // _KN])
            mout_v[pl.ds(g * 16, 16)] = jnp.where(
                tq == tn, jnp.float32(1.0), jnp.float32(0.0))
        pltpu.sync_copy(out_v, sumsq_hbm.at[pl.ds(p0, _PSUB)])
        pltpu.sync_copy(mout_v, match_hbm.at[pl.ds(p0, _PSUB)])
        return carry

    lax.fori_loop(0, _NSUB, sub_chunk, 0)


def _loss_body(s_ref, m_ref, out_ref):
    sumsq = s_ref[...]
    pos = m_ref[...]
    dist = jnp.sqrt(sumsq + _EPS)
    d = -dist
    d = d - jnp.max(d, axis=1, keepdims=True)
    e = jnp.exp(d)
    negs = jnp.sum(e * (1.0 - pos), axis=1, keepdims=True)
    under = e + negs
    cnt = jnp.sum(pos, axis=1, keepdims=True)
    point_mask = jnp.logical_and(cnt > 0.5, cnt < jnp.float32(_KN) - 0.5)
    valid = jnp.logical_and(pos > 0.5, point_mask)
    l = -jnp.log(e / under)
    lsum = jnp.sum(jnp.where(valid, l, 0.0), keepdims=True)
    vsum = jnp.sum(valid.astype(jnp.float32), keepdims=True)
    out_ref[...] = lsum / jnp.maximum(vsum, 1.0)


@jax.jit
def kernel(p, x, o, target):
    del o
    feats = pl.pallas_call(
        _normalize_body,
        out_shape=jax.ShapeDtypeStruct((_N, _CP), jnp.float32),
        interpret=_INTERPRET,
    )(jnp.pad(x, ((0, 0), (0, _CP - _C))))
    pp = jnp.pad(p, ((0, 0), (0, 5)))
    ppt = pp.T

    nblk = _N // _R
    idx_tab = pl.pallas_call(
        _knn_body,
        grid=(nblk,),
        in_specs=[
            pl.BlockSpec((_R, 8), lambda i: (i, 0)),
            pl.BlockSpec((8, _N), lambda i: (0, 0)),
        ],
        out_specs=pl.BlockSpec((_R, _KN), lambda i: (i, 0)),
        out_shape=jax.ShapeDtypeStruct((_N, _KN), jnp.int32),
        interpret=_INTERPRET,
    )(pp, ppt)

    idx2d = idx_tab.reshape(_NW * _NSUB, _PSUB)
    pairs = pl.kernel(
        _pairs_body,
        out_type=[
            jax.ShapeDtypeStruct((_P,), jnp.float32),
            jax.ShapeDtypeStruct((_P,), jnp.float32),
        ],
        mesh=plsc.VectorSubcoreMesh(core_axis_name="c", subcore_axis_name="s"),
        compiler_params=pltpu.CompilerParams(needs_layout_passes=False),
        scratch_types=[
            pltpu.VMEM((_NSUB, _PSUB), jnp.int32),
            pltpu.VMEM((_PSUB, _CP), jnp.float32),
            pltpu.VMEM((_QSUB, _CP), jnp.float32),
            pltpu.VMEM((_N,), jnp.int32),
            pltpu.VMEM((_PSUB * 16,), jnp.float32),
            pltpu.VMEM((_PSUB,), jnp.float32),
            pltpu.VMEM((_PSUB,), jnp.float32),
            pltpu.SemaphoreType.DMA,
        ],
    )
    sumsq_flat, match_flat = pairs(feats, idx2d, target)

    out = pl.pallas_call(
        _loss_body,
        out_shape=jax.ShapeDtypeStruct((1, 1), jnp.float32),
        interpret=_INTERPRET,
    )(sumsq_flat.reshape(_N, _KN), match_flat.reshape(_N, _KN))
    return out[0, 0]


# speedup vs baseline: 37.1521x; 1.0485x over previous
import functools

import jax
import jax.numpy as jnp
from jax import lax
from jax.experimental import pallas as pl
from jax.experimental.pallas import tpu as pltpu
from jax.experimental.pallas import tpu_sc as plsc

_N = 8192
_C = 64
_CP = 128
_K = 8
_KN = _K - 1
_EPS = 1e-12
_R = 256

_NW = 32
_QSUB = 16
_PSUB = _QSUB * _KN
_NSUB = _N // _NW // _QSUB
_P = _N * _KN

_INTERPRET = False


def _normalize_body(x_ref, f_ref):
    x = x_ref[...]
    norm = jnp.sqrt(jnp.sum(x * x, axis=-1, keepdims=True))
    f_ref[...] = x / jnp.maximum(norm, _EPS)


def _knn_body(pp_ref, ppt_ref, i_ref):
    pp = pp_ref[...]
    ppt = ppt_ref[...]
    pn_row = jnp.sum(ppt * ppt, axis=0, keepdims=True)
    pn_col = jnp.sum(pp * pp, axis=1, keepdims=True)
    dot = jnp.dot(pp, ppt, preferred_element_type=jnp.float32)
    d2 = pn_col + pn_row - 2.0 * dot

    iota = lax.broadcasted_iota(jnp.int32, (_R, _N), 1)
    s = lax.bitcast_convert_type(d2, jnp.int32)
    t = s ^ (lax.shift_right_arithmetic(s, 31) & jnp.int32(0x7FFFFFFF))
    cur = (t & jnp.int32(-8192)) | iota
    keys = []
    for _ in range(_K):
        mn = jnp.min(cur, axis=1, keepdims=True)
        keys.append(mn)
        cur = jnp.where(cur == mn, jnp.int32(0x7FFFFFFF), cur)
    i_ref[...] = jnp.concatenate(keys[1:], axis=1) & jnp.int32(8191)


def _pairs_body(feats_hbm, idx_hbm, tgt_hbm, sumsq_hbm, match_hbm,
                nidx_v, nf_v, qf_v, tgt_v, acc_v, out_v, mout_v, sem, qsem):
    wid = lax.axis_index("s") * 2 + lax.axis_index("c")
    qbase = wid * (_N // _NW)
    pltpu.sync_copy(tgt_hbm, tgt_v)
    pltpu.sync_copy(idx_hbm.at[pl.ds(wid * _NSUB, _NSUB)], nidx_v)
    lane = lax.broadcasted_iota(jnp.int32, (16,), 0)

    pltpu.async_copy(feats_hbm.at[nidx_v.at[0]], nf_v.at[0], sem)
    pltpu.async_copy(feats_hbm.at[pl.ds(qbase, _QSUB)], qf_v.at[0], qsem)

    def sub_chunk(s, carry):
        b = lax.rem(s, 2)
        q0 = qbase + s * _QSUB
        p0 = (q0 * _KN).astype(jnp.int32)
        pltpu.make_async_copy(feats_hbm.at[nidx_v.at[s]],
                              nf_v.at[b], sem).wait()
        pltpu.make_async_copy(feats_hbm.at[pl.ds(q0, _QSUB)],
                              qf_v.at[b], qsem).wait()

        @pl.when(s < _NSUB - 1)
        def _():
            pltpu.async_copy(feats_hbm.at[nidx_v.at[s + 1]],
                             nf_v.at[1 - b], sem)
            pltpu.async_copy(feats_hbm.at[pl.ds(q0 + _QSUB, _QSUB)],
                             qf_v.at[1 - b], qsem)

        for q in range(_QSUB):
            for j in range(_KN):
                r = q * _KN + j
                acc = jnp.zeros((16,), jnp.float32)
                for c in range(_C // 16):
                    d = (qf_v[b, q, pl.ds(c * 16, 16)]
                         - nf_v[b, r, pl.ds(c * 16, 16)])
                    acc = acc + d * d
                acc_v[pl.ds(r * 16, 16)] = acc
        for g in range(_PSUB // 16):
            tot = jnp.zeros((16,), jnp.float32)
            for c in range(16):
                tot = tot + plsc.load_gather(
                    acc_v,
                    [(jnp.full((16,), g * 16, jnp.int32) + lane) * 16
                     + jnp.full((16,), c, jnp.int32)])
            out_v[pl.ds(g * 16, 16)] = tot
            lp = jnp.full((16,), g * 16, jnp.int32) + lane
            nid = nidx_v[s, pl.ds(g * 16, 16)]
            tn = plsc.load_gather(tgt_v, [nid])
            tq = plsc.load_gather(tgt_v, [q0 + lp // _KN])
            mout_v[pl.ds(g * 16, 16)] = jnp.where(
                tq == tn, jnp.float32(1.0), jnp.float32(0.0))
        pltpu.sync_copy(out_v, sumsq_hbm.at[pl.ds(p0, _PSUB)])
        pltpu.sync_copy(mout_v, match_hbm.at[pl.ds(p0, _PSUB)])
        return carry

    lax.fori_loop(0, _NSUB, sub_chunk, 0)


def _loss_body(s_ref, m_ref, out_ref):
    sumsq = s_ref[...]
    pos = m_ref[...]
    dist = jnp.sqrt(sumsq + _EPS)
    d = -dist
    d = d - jnp.max(d, axis=1, keepdims=True)
    e = jnp.exp(d)
    negs = jnp.sum(e * (1.0 - pos), axis=1, keepdims=True)
    under = e + negs
    cnt = jnp.sum(pos, axis=1, keepdims=True)
    point_mask = jnp.logical_and(cnt > 0.5, cnt < jnp.float32(_KN) - 0.5)
    valid = jnp.logical_and(pos > 0.5, point_mask)
    l = -jnp.log(e / under)
    lsum = jnp.sum(jnp.where(valid, l, 0.0), keepdims=True)
    vsum = jnp.sum(valid.astype(jnp.float32), keepdims=True)
    out_ref[...] = lsum / jnp.maximum(vsum, 1.0)


@jax.jit
def kernel(p, x, o, target):
    del o
    feats = pl.pallas_call(
        _normalize_body,
        out_shape=jax.ShapeDtypeStruct((_N, _CP), jnp.float32),
        interpret=_INTERPRET,
    )(jnp.pad(x, ((0, 0), (0, _CP - _C))))
    pp = jnp.pad(p, ((0, 0), (0, 5)))
    ppt = pp.T

    nblk = _N // _R
    idx_tab = pl.pallas_call(
        _knn_body,
        grid=(nblk,),
        in_specs=[
            pl.BlockSpec((_R, 8), lambda i: (i, 0)),
            pl.BlockSpec((8, _N), lambda i: (0, 0)),
        ],
        out_specs=pl.BlockSpec((_R, _KN), lambda i: (i, 0)),
        out_shape=jax.ShapeDtypeStruct((_N, _KN), jnp.int32),
        interpret=_INTERPRET,
    )(pp, ppt)

    idx2d = idx_tab.reshape(_NW * _NSUB, _PSUB)
    pairs = pl.kernel(
        _pairs_body,
        out_type=[
            jax.ShapeDtypeStruct((_P,), jnp.float32),
            jax.ShapeDtypeStruct((_P,), jnp.float32),
        ],
        mesh=plsc.VectorSubcoreMesh(core_axis_name="c", subcore_axis_name="s"),
        compiler_params=pltpu.CompilerParams(needs_layout_passes=False),
        scratch_types=[
            pltpu.VMEM((_NSUB, _PSUB), jnp.int32),
            pltpu.VMEM((2, _PSUB, _CP), jnp.float32),
            pltpu.VMEM((2, _QSUB, _CP), jnp.float32),
            pltpu.VMEM((_N,), jnp.int32),
            pltpu.VMEM((_PSUB * 16,), jnp.float32),
            pltpu.VMEM((_PSUB,), jnp.float32),
            pltpu.VMEM((_PSUB,), jnp.float32),
            pltpu.SemaphoreType.DMA,
            pltpu.SemaphoreType.DMA,
        ],
    )
    sumsq_flat, match_flat = pairs(feats, idx2d, target)

    out = pl.pallas_call(
        _loss_body,
        out_shape=jax.ShapeDtypeStruct((1, 1), jnp.float32),
        interpret=_INTERPRET,
    )(sumsq_flat.reshape(_N, _KN), match_flat.reshape(_N, _KN))
    return out[0, 0]


# speedup vs baseline: 37.1947x; 1.0011x over previous
import functools

import jax
import jax.numpy as jnp
from jax import lax
from jax.experimental import pallas as pl
from jax.experimental.pallas import tpu as pltpu
from jax.experimental.pallas import tpu_sc as plsc

_N = 8192
_C = 64
_CP = 128
_K = 8
_KN = _K - 1
_EPS = 1e-12
_R = 256

_NW = 32
_QSUB = 16
_PSUB = _QSUB * _KN
_NSUB = _N // _NW // _QSUB
_P = _N * _KN

_INTERPRET = False


def _knn_body(pp_ref, ppt_ref, x_ref, i_ref, f_ref):
    x = x_ref[...]
    norm = jnp.sqrt(jnp.sum(x * x, axis=-1, keepdims=True))
    f_ref[...] = x / jnp.maximum(norm, _EPS)
    pp = pp_ref[...]
    ppt = ppt_ref[...]
    pn_row = jnp.sum(ppt * ppt, axis=0, keepdims=True)
    pn_col = jnp.sum(pp * pp, axis=1, keepdims=True)
    dot = jnp.dot(pp, ppt, preferred_element_type=jnp.float32)
    d2 = pn_col + pn_row - 2.0 * dot

    iota = lax.broadcasted_iota(jnp.int32, (_R, _N), 1)
    s = lax.bitcast_convert_type(d2, jnp.int32)
    t = s ^ (lax.shift_right_arithmetic(s, 31) & jnp.int32(0x7FFFFFFF))
    cur = (t & jnp.int32(-8192)) | iota
    keys = []
    for _ in range(_K):
        mn = jnp.min(cur, axis=1, keepdims=True)
        keys.append(mn)
        cur = jnp.where(cur == mn, jnp.int32(0x7FFFFFFF), cur)
    i_ref[...] = jnp.concatenate(keys[1:], axis=1) & jnp.int32(8191)


def _pairs_body(feats_hbm, idx_hbm, tgt_hbm, sumsq_hbm, match_hbm,
                nidx_v, nf_v, qf_v, tgt_v, acc_v, out_v, mout_v, sem, qsem):
    wid = lax.axis_index("s") * 2 + lax.axis_index("c")
    qbase = wid * (_N // _NW)
    pltpu.sync_copy(tgt_hbm, tgt_v)
    pltpu.sync_copy(idx_hbm.at[pl.ds(wid * _NSUB, _NSUB)], nidx_v)
    lane = lax.broadcasted_iota(jnp.int32, (16,), 0)

    pltpu.async_copy(feats_hbm.at[nidx_v.at[0]], nf_v.at[0], sem)
    pltpu.async_copy(feats_hbm.at[pl.ds(qbase, _QSUB)], qf_v.at[0], qsem)

    def sub_chunk(s, carry):
        b = lax.rem(s, 2)
        q0 = qbase + s * _QSUB
        p0 = (q0 * _KN).astype(jnp.int32)
        pltpu.make_async_copy(feats_hbm.at[nidx_v.at[s]],
                              nf_v.at[b], sem).wait()
        pltpu.make_async_copy(feats_hbm.at[pl.ds(q0, _QSUB)],
                              qf_v.at[b], qsem).wait()

        @pl.when(s < _NSUB - 1)
        def _():
            pltpu.async_copy(feats_hbm.at[nidx_v.at[s + 1]],
                             nf_v.at[1 - b], sem)
            pltpu.async_copy(feats_hbm.at[pl.ds(q0 + _QSUB, _QSUB)],
                             qf_v.at[1 - b], qsem)

        for q in range(_QSUB):
            for j in range(_KN):
                r = q * _KN + j
                acc = jnp.zeros((16,), jnp.float32)
                for c in range(_C // 16):
                    d = (qf_v[b, q, pl.ds(c * 16, 16)]
                         - nf_v[b, r, pl.ds(c * 16, 16)])
                    acc = acc + d * d
                acc_v[pl.ds(r * 16, 16)] = acc
        for g in range(_PSUB // 16):
            tot = jnp.zeros((16,), jnp.float32)
            for c in range(16):
                tot = tot + plsc.load_gather(
                    acc_v,
                    [(jnp.full((16,), g * 16, jnp.int32) + lane) * 16
                     + jnp.full((16,), c, jnp.int32)])
            out_v[pl.ds(g * 16, 16)] = tot
            lp = jnp.full((16,), g * 16, jnp.int32) + lane
            nid = nidx_v[s, pl.ds(g * 16, 16)]
            tn = plsc.load_gather(tgt_v, [nid])
            tq = plsc.load_gather(tgt_v, [q0 + lp // _KN])
            mout_v[pl.ds(g * 16, 16)] = jnp.where(
                tq == tn, jnp.float32(1.0), jnp.float32(0.0))
        pltpu.sync_copy(out_v, sumsq_hbm.at[pl.ds(p0, _PSUB)])
        pltpu.sync_copy(mout_v, match_hbm.at[pl.ds(p0, _PSUB)])
        return carry

    lax.fori_loop(0, _NSUB, sub_chunk, 0)


def _loss_body(s_ref, m_ref, out_ref):
    sumsq = s_ref[...]
    pos = m_ref[...]
    dist = jnp.sqrt(sumsq + _EPS)
    d = -dist
    d = d - jnp.max(d, axis=1, keepdims=True)
    e = jnp.exp(d)
    negs = jnp.sum(e * (1.0 - pos), axis=1, keepdims=True)
    under = e + negs
    cnt = jnp.sum(pos, axis=1, keepdims=True)
    point_mask = jnp.logical_and(cnt > 0.5, cnt < jnp.float32(_KN) - 0.5)
    valid = jnp.logical_and(pos > 0.5, point_mask)
    l = -jnp.log(e / under)
    lsum = jnp.sum(jnp.where(valid, l, 0.0), keepdims=True)
    vsum = jnp.sum(valid.astype(jnp.float32), keepdims=True)
    out_ref[...] = lsum / jnp.maximum(vsum, 1.0)


@jax.jit
def kernel(p, x, o, target):
    del o
    pp = jnp.pad(p, ((0, 0), (0, 5)))
    ppt = pp.T
    xp = jnp.pad(x, ((0, 0), (0, _CP - _C)))

    nblk = _N // _R
    idx_tab, feats = pl.pallas_call(
        _knn_body,
        grid=(nblk,),
        in_specs=[
            pl.BlockSpec((_R, 8), lambda i: (i, 0)),
            pl.BlockSpec((8, _N), lambda i: (0, 0)),
            pl.BlockSpec((_R, _CP), lambda i: (i, 0)),
        ],
        out_specs=[
            pl.BlockSpec((_R, _KN), lambda i: (i, 0)),
            pl.BlockSpec((_R, _CP), lambda i: (i, 0)),
        ],
        out_shape=[
            jax.ShapeDtypeStruct((_N, _KN), jnp.int32),
            jax.ShapeDtypeStruct((_N, _CP), jnp.float32),
        ],
        interpret=_INTERPRET,
    )(pp, ppt, xp)

    idx2d = idx_tab.reshape(_NW * _NSUB, _PSUB)
    pairs = pl.kernel(
        _pairs_body,
        out_type=[
            jax.ShapeDtypeStruct((_P,), jnp.float32),
            jax.ShapeDtypeStruct((_P,), jnp.float32),
        ],
        mesh=plsc.VectorSubcoreMesh(core_axis_name="c", subcore_axis_name="s"),
        compiler_params=pltpu.CompilerParams(needs_layout_passes=False),
        scratch_types=[
            pltpu.VMEM((_NSUB, _PSUB), jnp.int32),
            pltpu.VMEM((2, _PSUB, _CP), jnp.float32),
            pltpu.VMEM((2, _QSUB, _CP), jnp.float32),
            pltpu.VMEM((_N,), jnp.int32),
            pltpu.VMEM((_PSUB * 16,), jnp.float32),
            pltpu.VMEM((_PSUB,), jnp.float32),
            pltpu.VMEM((_PSUB,), jnp.float32),
            pltpu.SemaphoreType.DMA,
            pltpu.SemaphoreType.DMA,
        ],
    )
    sumsq_flat, match_flat = pairs(feats, idx2d, target)

    out = pl.pallas_call(
        _loss_body,
        out_shape=jax.ShapeDtypeStruct((1, 1), jnp.float32),
        interpret=_INTERPRET,
    )(sumsq_flat.reshape(_N, _KN), match_flat.reshape(_N, _KN))
    return out[0, 0]


# speedup vs baseline: 47.6990x; 1.2824x over previous
import functools

import jax
import jax.numpy as jnp
from jax import lax
from jax.experimental import pallas as pl
from jax.experimental.pallas import tpu as pltpu
from jax.experimental.pallas import tpu_sc as plsc

_N = 8192
_C = 64
_CP = 128
_K = 8
_KN = _K - 1
_EPS = 1e-12
_R = 256

_NW = 32
_QSUB = 16
_PSUB = _QSUB * _KN
_NSUB = _N // _NW // _QSUB
_P = _N * _KN

_INTERPRET = False


def _knn_body(pp_ref, ppt_ref, x_ref, i_ref, f_ref):
    x = x_ref[...]
    norm = jnp.sqrt(jnp.sum(x * x, axis=-1, keepdims=True))
    f_ref[...] = x / jnp.maximum(norm, _EPS)
    pp = pp_ref[...]
    ppt = ppt_ref[...]
    pn_row = jnp.sum(ppt * ppt, axis=0, keepdims=True)
    pn_col = jnp.sum(pp * pp, axis=1, keepdims=True)
    dot = jnp.dot(pp, ppt, preferred_element_type=jnp.float32)
    d2 = pn_col + pn_row - 2.0 * dot

    iota = lax.broadcasted_iota(jnp.int32, (_R, _N), 1)
    s = lax.bitcast_convert_type(d2, jnp.int32)
    t = s ^ (lax.shift_right_arithmetic(s, 31) & jnp.int32(0x7FFFFFFF))
    cur = lax.bitcast_convert_type((t & jnp.int32(-8192)) | iota,
                                   jnp.float32)
    keys = []
    for _ in range(_K):
        mn = jnp.min(cur, axis=1, keepdims=True)
        keys.append(mn)
        cur = jnp.where(cur == mn, jnp.float32(3e38), cur)
    i_ref[...] = (lax.bitcast_convert_type(
        jnp.concatenate(keys[1:], axis=1), jnp.int32) & jnp.int32(8191))


def _pairs_body(feats_hbm, idx_hbm, tgt_hbm, sumsq_hbm, match_hbm,
                nidx_v, nf_v, qf_v, tgt_v, acc_v, out_v, mout_v, sem, qsem):
    wid = lax.axis_index("s") * 2 + lax.axis_index("c")
    qbase = wid * (_N // _NW)
    pltpu.sync_copy(tgt_hbm, tgt_v)
    pltpu.sync_copy(idx_hbm.at[pl.ds(wid * _NSUB, _NSUB)], nidx_v)
    lane = lax.broadcasted_iota(jnp.int32, (16,), 0)

    pltpu.async_copy(feats_hbm.at[nidx_v.at[0]], nf_v.at[0], sem)
    pltpu.async_copy(feats_hbm.at[pl.ds(qbase, _QSUB)], qf_v.at[0], qsem)

    def sub_chunk(s, carry):
        b = lax.rem(s, 2)
        q0 = qbase + s * _QSUB
        p0 = (q0 * _KN).astype(jnp.int32)
        pltpu.make_async_copy(feats_hbm.at[nidx_v.at[s]],
                              nf_v.at[b], sem).wait()
        pltpu.make_async_copy(feats_hbm.at[pl.ds(q0, _QSUB)],
                              qf_v.at[b], qsem).wait()

        @pl.when(s < _NSUB - 1)
        def _():
            pltpu.async_copy(feats_hbm.at[nidx_v.at[s + 1]],
                             nf_v.at[1 - b], sem)
            pltpu.async_copy(feats_hbm.at[pl.ds(q0 + _QSUB, _QSUB)],
                             qf_v.at[1 - b], qsem)

        for q in range(_QSUB):
            for j in range(_KN):
                r = q * _KN + j
                acc = jnp.zeros((16,), jnp.float32)
                for c in range(_C // 16):
                    d = (qf_v[b, q, pl.ds(c * 16, 16)]
                         - nf_v[b, r, pl.ds(c * 16, 16)])
                    acc = acc + d * d
                acc_v[pl.ds(r * 16, 16)] = acc
        for g in range(_PSUB // 16):
            tot = jnp.zeros((16,), jnp.float32)
            for c in range(16):
                tot = tot + plsc.load_gather(
                    acc_v,
                    [(jnp.full((16,), g * 16, jnp.int32) + lane) * 16
                     + jnp.full((16,), c, jnp.int32)])
            out_v[pl.ds(g * 16, 16)] = tot
            lp = jnp.full((16,), g * 16, jnp.int32) + lane
            nid = nidx_v[s, pl.ds(g * 16, 16)]
            tn = plsc.load_gather(tgt_v, [nid])
            tq = plsc.load_gather(tgt_v, [q0 + lp // _KN])
            mout_v[pl.ds(g * 16, 16)] = jnp.where(
                tq == tn, jnp.float32(1.0), jnp.float32(0.0))
        pltpu.sync_copy(out_v, sumsq_hbm.at[pl.ds(p0, _PSUB)])
        pltpu.sync_copy(mout_v, match_hbm.at[pl.ds(p0, _PSUB)])
        return carry

    lax.fori_loop(0, _NSUB, sub_chunk, 0)


def _loss_body(s_ref, m_ref, out_ref):
    sumsq = s_ref[...]
    pos = m_ref[...]
    dist = jnp.sqrt(sumsq + _EPS)
    d = -dist
    d = d - jnp.max(d, axis=1, keepdims=True)
    e = jnp.exp(d)
    negs = jnp.sum(e * (1.0 - pos), axis=1, keepdims=True)
    under = e + negs
    cnt = jnp.sum(pos, axis=1, keepdims=True)
    point_mask = jnp.logical_and(cnt > 0.5, cnt < jnp.float32(_KN) - 0.5)
    valid = jnp.logical_and(pos > 0.5, point_mask)
    l = -jnp.log(e / under)
    lsum = jnp.sum(jnp.where(valid, l, 0.0), keepdims=True)
    vsum = jnp.sum(valid.astype(jnp.float32), keepdims=True)
    out_ref[...] = lsum / jnp.maximum(vsum, 1.0)


@jax.jit
def kernel(p, x, o, target):
    del o
    pp = jnp.pad(p, ((0, 0), (0, 5)))
    ppt = pp.T
    xp = jnp.pad(x, ((0, 0), (0, _CP - _C)))

    nblk = _N // _R
    idx_tab, feats = pl.pallas_call(
        _knn_body,
        grid=(nblk,),
        in_specs=[
            pl.BlockSpec((_R, 8), lambda i: (i, 0)),
            pl.BlockSpec((8, _N), lambda i: (0, 0)),
            pl.BlockSpec((_R, _CP), lambda i: (i, 0)),
        ],
        out_specs=[
            pl.BlockSpec((_R, _KN), lambda i: (i, 0)),
            pl.BlockSpec((_R, _CP), lambda i: (i, 0)),
        ],
        out_shape=[
            jax.ShapeDtypeStruct((_N, _KN), jnp.int32),
            jax.ShapeDtypeStruct((_N, _CP), jnp.float32),
        ],
        interpret=_INTERPRET,
    )(pp, ppt, xp)

    idx2d = idx_tab.reshape(_NW * _NSUB, _PSUB)
    pairs = pl.kernel(
        _pairs_body,
        out_type=[
            jax.ShapeDtypeStruct((_P,), jnp.float32),
            jax.ShapeDtypeStruct((_P,), jnp.float32),
        ],
        mesh=plsc.VectorSubcoreMesh(core_axis_name="c", subcore_axis_name="s"),
        compiler_params=pltpu.CompilerParams(needs_layout_passes=False),
        scratch_types=[
            pltpu.VMEM((_NSUB, _PSUB), jnp.int32),
            pltpu.VMEM((2, _PSUB, _CP), jnp.float32),
            pltpu.VMEM((2, _QSUB, _CP), jnp.float32),
            pltpu.VMEM((_N,), jnp.int32),
            pltpu.VMEM((_PSUB * 16,), jnp.float32),
            pltpu.VMEM((_PSUB,), jnp.float32),
            pltpu.VMEM((_PSUB,), jnp.float32),
            pltpu.SemaphoreType.DMA,
            pltpu.SemaphoreType.DMA,
        ],
    )
    sumsq_flat, match_flat = pairs(feats, idx2d, target)

    out = pl.pallas_call(
        _loss_body,
        out_shape=jax.ShapeDtypeStruct((1, 1), jnp.float32),
        interpret=_INTERPRET,
    )(sumsq_flat.reshape(_N, _KN), match_flat.reshape(_N, _KN))
    return out[0, 0]


# speedup vs baseline: 48.5301x; 1.0174x over previous
import functools

import jax
import jax.numpy as jnp
from jax import lax
from jax.experimental import pallas as pl
from jax.experimental.pallas import tpu as pltpu
from jax.experimental.pallas import tpu_sc as plsc

_N = 8192
_C = 64
_CP = 128
_K = 8
_KN = _K - 1
_EPS = 1e-12
_R = 256

_NW = 32
_QSUB = 16
_PSUB = _QSUB * _KN
_NSUB = _N // _NW // _QSUB
_P = _N * _KN

_INTERPRET = False


def _knn_body(pp_ref, ppt_ref, x_ref, i_ref, f_ref):
    x = x_ref[...]
    norm = jnp.sqrt(jnp.sum(x * x, axis=-1, keepdims=True))
    f_ref[...] = x / jnp.maximum(norm, _EPS)
    pp = pp_ref[...]
    ppt = ppt_ref[...]
    pn_row = jnp.sum(ppt * ppt, axis=0, keepdims=True)
    pn_col = jnp.sum(pp * pp, axis=1, keepdims=True)
    dot = jnp.dot(pp, ppt, preferred_element_type=jnp.float32)
    d2 = pn_col + pn_row - 2.0 * dot

    iota = lax.broadcasted_iota(jnp.int32, (_R, _N), 1)
    dz = jnp.where(d2 == 0.0, jnp.float32(1.2e-38), d2)
    cur = lax.bitcast_convert_type(
        (lax.bitcast_convert_type(dz, jnp.int32) & jnp.int32(-8192)) | iota,
        jnp.float32)
    keys = []
    for _ in range(_K):
        mn = jnp.min(cur, axis=1, keepdims=True)
        keys.append(mn)
        cur = jnp.where(cur == mn, jnp.float32(3e38), cur)
    i_ref[...] = (lax.bitcast_convert_type(
        jnp.concatenate(keys[1:], axis=1), jnp.int32) & jnp.int32(8191))


def _pairs_body(feats_hbm, idx_hbm, tgt_hbm, sumsq_hbm, match_hbm,
                nidx_v, nf_v, qf_v, tgt_v, acc_v, out_v, mout_v, sem, qsem):
    wid = lax.axis_index("s") * 2 + lax.axis_index("c")
    qbase = wid * (_N // _NW)
    pltpu.sync_copy(tgt_hbm, tgt_v)
    pltpu.sync_copy(idx_hbm.at[pl.ds(wid * _NSUB, _NSUB)], nidx_v)
    lane = lax.broadcasted_iota(jnp.int32, (16,), 0)

    pltpu.async_copy(feats_hbm.at[nidx_v.at[0]], nf_v.at[0], sem)
    pltpu.async_copy(feats_hbm.at[pl.ds(qbase, _QSUB)], qf_v.at[0], qsem)

    def sub_chunk(s, carry):
        b = lax.rem(s, 2)
        q0 = qbase + s * _QSUB
        p0 = (q0 * _KN).astype(jnp.int32)
        pltpu.make_async_copy(feats_hbm.at[nidx_v.at[s]],
                              nf_v.at[b], sem).wait()
        pltpu.make_async_copy(feats_hbm.at[pl.ds(q0, _QSUB)],
                              qf_v.at[b], qsem).wait()

        @pl.when(s < _NSUB - 1)
        def _():
            pltpu.async_copy(feats_hbm.at[nidx_v.at[s + 1]],
                             nf_v.at[1 - b], sem)
            pltpu.async_copy(feats_hbm.at[pl.ds(q0 + _QSUB, _QSUB)],
                             qf_v.at[1 - b], qsem)

        for q in range(_QSUB):
            for j in range(_KN):
                r = q * _KN + j
                acc = jnp.zeros((16,), jnp.float32)
                for c in range(_C // 16):
                    d = (qf_v[b, q, pl.ds(c * 16, 16)]
                         - nf_v[b, r, pl.ds(c * 16, 16)])
                    acc = acc + d * d
                acc_v[pl.ds(r * 16, 16)] = acc
        for g in range(_PSUB // 16):
            tot = jnp.zeros((16,), jnp.float32)
            for c in range(16):
                tot = tot + plsc.load_gather(
                    acc_v,
                    [(jnp.full((16,), g * 16, jnp.int32) + lane) * 16
                     + jnp.full((16,), c, jnp.int32)])
            out_v[pl.ds(g * 16, 16)] = tot
            lp = jnp.full((16,), g * 16, jnp.int32) + lane
            nid = nidx_v[s, pl.ds(g * 16, 16)]
            tn = plsc.load_gather(tgt_v, [nid])
            tq = plsc.load_gather(tgt_v, [q0 + lp // _KN])
            mout_v[pl.ds(g * 16, 16)] = jnp.where(
                tq == tn, jnp.float32(1.0), jnp.float32(0.0))
        pltpu.sync_copy(out_v, sumsq_hbm.at[pl.ds(p0, _PSUB)])
        pltpu.sync_copy(mout_v, match_hbm.at[pl.ds(p0, _PSUB)])
        return carry

    lax.fori_loop(0, _NSUB, sub_chunk, 0)


def _loss_body(s_ref, m_ref, out_ref):
    sumsq = s_ref[...]
    pos = m_ref[...]
    dist = jnp.sqrt(sumsq + _EPS)
    d = -dist
    d = d - jnp.max(d, axis=1, keepdims=True)
    e = jnp.exp(d)
    negs = jnp.sum(e * (1.0 - pos), axis=1, keepdims=True)
    under = e + negs
    cnt = jnp.sum(pos, axis=1, keepdims=True)
    point_mask = jnp.logical_and(cnt > 0.5, cnt < jnp.float32(_KN) - 0.5)
    valid = jnp.logical_and(pos > 0.5, point_mask)
    l = -jnp.log(e / under)
    lsum = jnp.sum(jnp.where(valid, l, 0.0), keepdims=True)
    vsum = jnp.sum(valid.astype(jnp.float32), keepdims=True)
    out_ref[...] = lsum / jnp.maximum(vsum, 1.0)


@jax.jit
def kernel(p, x, o, target):
    del o
    pp = jnp.pad(p, ((0, 0), (0, 5)))
    ppt = pp.T
    xp = jnp.pad(x, ((0, 0), (0, _CP - _C)))

    nblk = _N // _R
    idx_tab, feats = pl.pallas_call(
        _knn_body,
        grid=(nblk,),
        in_specs=[
            pl.BlockSpec((_R, 8), lambda i: (i, 0)),
            pl.BlockSpec((8, _N), lambda i: (0, 0)),
            pl.BlockSpec((_R, _CP), lambda i: (i, 0)),
        ],
        out_specs=[
            pl.BlockSpec((_R, _KN), lambda i: (i, 0)),
            pl.BlockSpec((_R, _CP), lambda i: (i, 0)),
        ],
        out_shape=[
            jax.ShapeDtypeStruct((_N, _KN), jnp.int32),
            jax.ShapeDtypeStruct((_N, _CP), jnp.float32),
        ],
        interpret=_INTERPRET,
    )(pp, ppt, xp)

    idx2d = idx_tab.reshape(_NW * _NSUB, _PSUB)
    pairs = pl.kernel(
        _pairs_body,
        out_type=[
            jax.ShapeDtypeStruct((_P,), jnp.float32),
            jax.ShapeDtypeStruct((_P,), jnp.float32),
        ],
        mesh=plsc.VectorSubcoreMesh(core_axis_name="c", subcore_axis_name="s"),
        compiler_params=pltpu.CompilerParams(needs_layout_passes=False),
        scratch_types=[
            pltpu.VMEM((_NSUB, _PSUB), jnp.int32),
            pltpu.VMEM((2, _PSUB, _CP), jnp.float32),
            pltpu.VMEM((2, _QSUB, _CP), jnp.float32),
            pltpu.VMEM((_N,), jnp.int32),
            pltpu.VMEM((_PSUB * 16,), jnp.float32),
            pltpu.VMEM((_PSUB,), jnp.float32),
            pltpu.VMEM((_PSUB,), jnp.float32),
            pltpu.SemaphoreType.DMA,
            pltpu.SemaphoreType.DMA,
        ],
    )
    sumsq_flat, match_flat = pairs(feats, idx2d, target)

    out = pl.pallas_call(
        _loss_body,
        out_shape=jax.ShapeDtypeStruct((1, 1), jnp.float32),
        interpret=_INTERPRET,
    )(sumsq_flat.reshape(_N, _KN), match_flat.reshape(_N, _KN))
    return out[0, 0]
